# baseline (speedup 1.0000x reference)
import jax
import jax.numpy as jnp
from jax import lax
from jax.experimental import pallas as pl
from jax.experimental.pallas import tpu as pltpu
from jax.experimental.pallas import tpu_sc as plsc

N = 10000
E = 320000
H = 768
F1 = 128
F2 = 16
NC = 2
NS = 16
NT = NC * NS
K = 80
NCHUNK = (E // NT) // K
NCHUNK2 = (E // NS) // K
NPAD = 10240
RPT = NPAD // NS
NHALF = NPAD // 2
FH = F1 // NC
NBUF = 5

_mesh = plsc.VectorSubcoreMesh(core_axis_name="c", subcore_axis_name="s",
                               num_cores=NC)


def _deg_body(col_hbm, ew_hbm, out_hbm, col_v, ew_v, zb_v, deg_sp):
    c = lax.axis_index("c")
    s = lax.axis_index("s")
    zsl = NPAD // NS

    def zb(i, carry):
        zb_v[pl.ds(i * 16, 16)] = jnp.zeros((16,), jnp.float32)
        return carry

    lax.fori_loop(0, zsl // 16, zb, 0)
    pltpu.sync_copy(zb_v, deg_sp.at[pl.ds(s * zsl, zsl)])
    plsc.subcore_barrier()

    wid = c * NS + s
    pltpu.sync_copy(col_hbm.at[wid], col_v)
    pltpu.sync_copy(ew_hbm.at[wid], ew_v)

    def chunk(j, carry):
        pltpu.sync_copy(ew_v.at[j], deg_sp.at[col_v.at[j]], add=True)
        return carry

    lax.fori_loop(0, NCHUNK, chunk, 0)
    plsc.subcore_barrier()
    pltpu.sync_copy(deg_sp.at[pl.ds(s * zsl, zsl)],
                    out_hbm.at[c, pl.ds(s * zsl, zsl)])


_deg = pl.kernel(
    _deg_body,
    out_type=jax.ShapeDtypeStruct((NC, NPAD), jnp.float32),
    mesh=_mesh,
    compiler_params=pltpu.CompilerParams(use_tc_tiling_on_sc=False),
    scratch_types=[
        pltpu.VMEM((NCHUNK, K), jnp.int32),
        pltpu.VMEM((NCHUNK, K), jnp.float32),
        pltpu.VMEM((NPAD // NS,), jnp.float32),
        pltpu.VMEM_SHARED((NPAD,), jnp.float32),
    ],
)


def _scale_chunk(rows_v, ew_v, j, F):
    def grp(g, carry):
        wv = ew_v[j, pl.ds(g * 16, 16)]
        for l in range(16):
            w = wv[l]
            e = g * 16 + l
            for f in range(F // 16):
                sl = pl.ds(f * 16, 16)
                rows_v[e, sl] = rows_v[e, sl] * w
        return carry

    lax.fori_loop(0, K // 16, grp, 0)


def _make_agg_body(F, nchunk, mode):
    nq = nchunk // NBUF
    fsplit = mode == "fsplit"
    rpt = RPT if fsplit else NHALF // NS
    zch = rpt // 5

    def body(z_hbm, row_hbm, col_hbm, ew_hbm, out_hbm,
             row_v, col_v, ew_v, r0, r1, zb_v, acc_sp, g0, g1):
        c = lax.axis_index("c")
        s = lax.axis_index("s")
        rows = [r0, r1]
        gsem = [g0, g1]

        def zb(i, carry):
            for f in range(F // 16):
                zb_v[i, pl.ds(f * 16, 16)] = jnp.zeros((16,), jnp.float32)
            return carry

        lax.fori_loop(0, zch, zb, 0)
        for t in range(5):
            pltpu.sync_copy(zb_v, acc_sp.at[pl.ds(s * rpt + t * zch, zch)])
        plsc.subcore_barrier()

        pltpu.sync_copy(row_hbm.at[s], row_v)
        pltpu.sync_copy(col_hbm.at[s], col_v)
        pltpu.sync_copy(ew_hbm.at[s], ew_v)

        if not fsplit:
            base = c * NHALF

            def fixrow(jj, carry):
                for g in range(K // 16):
                    sl = pl.ds(g * 16, 16)
                    v = col_v[jj, sl] - base
                    ok = (v >= 0) & (v < NHALF)
                    col_v[jj, sl] = jnp.where(ok, v, v & 4095)
                    wv = ew_v[jj, sl]
                    ew_v[jj, sl] = jnp.where(ok, wv, 0.0)
                return carry

            lax.fori_loop(0, nchunk, fixrow, 0)

        def zsrc(j):
            if fsplit:
                return z_hbm.at[c].at[row_v.at[j]]
            return z_hbm.at[row_v.at[j]]

        def chunk(j, carry):
            pltpu.async_copy(zsrc(j), rows[0], gsem[0]).wait()
            _scale_chunk(rows[0], ew_v, j, F)
            pltpu.sync_copy(rows[0], acc_sp.at[col_v.at[j]], add=True)
            return carry

        lax.fori_loop(0, nchunk, chunk, 0)
        plsc.subcore_barrier()
        for t in range(5):
            r0w = s * rpt + t * zch
            if fsplit:
                pltpu.sync_copy(acc_sp.at[pl.ds(r0w, zch)],
                                out_hbm.at[c, pl.ds(r0w, zch)])
            else:
                pltpu.sync_copy(acc_sp.at[pl.ds(r0w, zch)],
                                out_hbm.at[pl.ds(c * NHALF + r0w, zch)])

    return body


def _make_agg(F, nchunk, mode):
    fsplit = mode == "fsplit"
    out_sh = (NC, NPAD, F) if fsplit else (NPAD, F)
    acc_rows = NPAD if fsplit else NHALF
    zch = (RPT if fsplit else NHALF // NS) // 5
    return pl.kernel(
        _make_agg_body(F, nchunk, mode),
        out_type=jax.ShapeDtypeStruct(out_sh, jnp.float32),
        mesh=_mesh,
        compiler_params=pltpu.CompilerParams(use_tc_tiling_on_sc=False),
        scratch_types=[
            pltpu.VMEM((nchunk, K), jnp.int32),
            pltpu.VMEM((nchunk, K), jnp.int32),
            pltpu.VMEM((nchunk, K), jnp.float32),
        ] + [pltpu.VMEM((K, F), jnp.float32)] * 2 + [
            pltpu.VMEM((zch, F), jnp.float32),
            pltpu.VMEM_SHARED((acc_rows, F), jnp.float32),
        ] + [pltpu.SemaphoreType.DMA] * 2,
    )


_agg128 = _make_agg(FH, NCHUNK2, "fsplit")
_agg16 = _make_agg(F2, NCHUNK2, "nsplit")


MBLK = 1000


def _tc_a_body(emb, dwT, db, w1aT, w1b, w1c, pn, tl, d0, d1,
               z1_out, z1h_out, dis_out):
    t = jnp.tanh(jnp.dot(emb[...], dwT[...],
                         preferred_element_type=jnp.float32,
                         precision=lax.Precision.HIGHEST) + db[...])
    y = jnp.dot(t, w1aT[...], preferred_element_type=jnp.float32,
                precision=lax.Precision.HIGHEST)
    y = y + pn[...] * w1b[...] + tl[...] * w1c[...]
    deg = d0[...] + d1[...] + 2.0
    dis = jnp.where(deg > 0, lax.rsqrt(deg), 0.0)
    z = dis * y
    z1_out[...] = z
    z1h_out[0, :, :] = z[:, :FH]
    z1h_out[1, :, :] = z[:, FH:]
    dis_out[...] = dis


def _tc_a(emb, dwT, db, w1aT, w1b, w1c, pn, tl, d0, d1):
    return pl.pallas_call(
        _tc_a_body,
        grid=(N // MBLK,),
        in_specs=[
            pl.BlockSpec((MBLK, H), lambda i: (i, 0)),
            pl.BlockSpec((H, H), lambda i: (0, 0)),
            pl.BlockSpec((1, H), lambda i: (0, 0)),
            pl.BlockSpec((H, F1), lambda i: (0, 0)),
            pl.BlockSpec((1, F1), lambda i: (0, 0)),
            pl.BlockSpec((1, F1), lambda i: (0, 0)),
            pl.BlockSpec((MBLK, 1), lambda i: (i, 0)),
            pl.BlockSpec((MBLK, 1), lambda i: (i, 0)),
            pl.BlockSpec((MBLK, 1), lambda i: (i, 0)),
            pl.BlockSpec((MBLK, 1), lambda i: (i, 0)),
        ],
        out_specs=[
            pl.BlockSpec((MBLK, F1), lambda i: (i, 0)),
            pl.BlockSpec((NC, MBLK, FH), lambda i: (0, i, 0)),
            pl.BlockSpec((MBLK, 1), lambda i: (i, 0)),
        ],
        out_shape=[
            jax.ShapeDtypeStruct((N, F1), jnp.float32),
            jax.ShapeDtypeStruct((NC, N, FH), jnp.float32),
            jax.ShapeDtypeStruct((N, 1), jnp.float32),
        ],
    )(emb, dwT, db, w1aT, w1b, w1c, pn, tl, d0, d1)


def _tc_e_body(acc, z1, dis, b1r, w2Tp, z2_out):
    a = jnp.concatenate([acc[0], acc[1]], axis=1)
    h = jnp.maximum(dis[...] * (a + 2.0 * z1[...]) + b1r[...], 0.0)
    y2 = jnp.dot(h, w2Tp[...], preferred_element_type=jnp.float32,
                 precision=lax.Precision.HIGHEST)
    z2_out[...] = dis[...] * y2


def _tc_e(acc, z1, dis, b1r, w2Tp):
    return pl.pallas_call(
        _tc_e_body,
        out_shape=jax.ShapeDtypeStruct((N, F2), jnp.float32),
    )(acc, z1, dis, b1r, w2Tp)


def _tc_g_body(acc, z2, dis, b2p, out):
    out[...] = dis[...] * (acc[...] + 2.0 * z2[...]) + b2p[...]


def _tc_g(acc, z2, dis, b2p):
    return pl.pallas_call(
        _tc_g_body,
        out_shape=jax.ShapeDtypeStruct((N, F2), jnp.float32),
    )(acc, z2, dis, b2p)


def kernel(embedding, p_num, text_len, edge_index, edge_attr,
           dense_w, dense_b, w1, b1, w2, b2):
    row3 = edge_index[0].reshape(NT, NCHUNK, K)
    col3 = edge_index[1].reshape(NT, NCHUNK, K)
    ew3 = edge_attr.reshape(NT, NCHUNK, K)
    row2 = edge_index[0].reshape(NS, NCHUNK2, K)
    col2 = edge_index[1].reshape(NS, NCHUNK2, K)
    ew2 = edge_attr.reshape(NS, NCHUNK2, K)

    deg_parts = _deg(col3, ew3)
    d0 = deg_parts[0, :N][:, None]
    d1 = deg_parts[1, :N][:, None]

    dwT = dense_w.T
    w1aT = w1[:, :H].T
    w1b = w1[:, H][None, :]
    w1c = w1[:, H + 1][None, :]
    z1, z1h, dis = _tc_a(embedding, dwT, dense_b[None, :], w1aT, w1b, w1c,
                         p_num, text_len, d0, d1)

    acc1 = _agg128(z1h, row2, col2, ew2)[:, :N, :]

    w2Tp = jnp.concatenate(
        [w2.T, jnp.zeros((F1, F2 - w2.shape[0]), jnp.float32)], axis=1)
    z2 = _tc_e(acc1, z1, dis, b1[None, :], w2Tp)

    acc2 = _agg16(z2, row2, col2, ew2)[:N, :]

    b2p = jnp.concatenate(
        [b2, jnp.zeros((F2 - b2.shape[0],), jnp.float32)])[None, :]
    out16 = _tc_g(acc2, z2, dis, b2p)
    return out16[:, :10]


# speedup vs baseline: 1.2927x; 1.2927x over previous
import jax
import jax.numpy as jnp
from jax import lax
from jax.experimental import pallas as pl
from jax.experimental.pallas import tpu as pltpu
from jax.experimental.pallas import tpu_sc as plsc

N = 10000
E = 320000
H = 768
F1 = 128
F2 = 16
NC = 2
NS = 16
NT = NC * NS
K = 80
NCHUNK = (E // NT) // K
NCHUNK2 = (E // NS) // K
NPAD = 10240
RPT = NPAD // NS
NHALF = NPAD // 2
FH = F1 // NC
NBUF = 4

_mesh = plsc.VectorSubcoreMesh(core_axis_name="c", subcore_axis_name="s",
                               num_cores=NC)


def _deg_body(col_hbm, ew_hbm, out_hbm, col_v, ew_v, zb_v, deg_sp):
    c = lax.axis_index("c")
    s = lax.axis_index("s")
    zsl = NPAD // NS

    def zb(i, carry):
        zb_v[pl.ds(i * 16, 16)] = jnp.zeros((16,), jnp.float32)
        return carry

    lax.fori_loop(0, zsl // 16, zb, 0)
    pltpu.sync_copy(zb_v, deg_sp.at[pl.ds(s * zsl, zsl)])
    plsc.subcore_barrier()

    wid = c * NS + s
    pltpu.sync_copy(col_hbm.at[wid], col_v)
    pltpu.sync_copy(ew_hbm.at[wid], ew_v)

    def chunk(j, carry):
        pltpu.sync_copy(ew_v.at[j], deg_sp.at[col_v.at[j]], add=True)
        return carry

    lax.fori_loop(0, NCHUNK, chunk, 0)
    plsc.subcore_barrier()
    pltpu.sync_copy(deg_sp.at[pl.ds(s * zsl, zsl)],
                    out_hbm.at[c, pl.ds(s * zsl, zsl)])


_deg = pl.kernel(
    _deg_body,
    out_type=jax.ShapeDtypeStruct((NC, NPAD), jnp.float32),
    mesh=_mesh,
    compiler_params=pltpu.CompilerParams(use_tc_tiling_on_sc=False),
    scratch_types=[
        pltpu.VMEM((NCHUNK, K), jnp.int32),
        pltpu.VMEM((NCHUNK, K), jnp.float32),
        pltpu.VMEM((NPAD // NS,), jnp.float32),
        pltpu.VMEM_SHARED((NPAD,), jnp.float32),
    ],
)


def _scale_chunk(rows_v, ew_v, j, F):
    def grp(g, carry):
        wv = ew_v[j, pl.ds(g * 16, 16)]
        for l in range(16):
            w = wv[l]
            e = g * 16 + l
            for f in range(F // 16):
                sl = pl.ds(f * 16, 16)
                rows_v[e, sl] = rows_v[e, sl] * w
        return carry

    lax.fori_loop(0, K // 16, grp, 0)


def _make_agg_body(F, nchunk, mode):
    nq = nchunk // NBUF
    fsplit = mode == "fsplit"
    rpt = RPT if fsplit else NHALF // NS
    zch = rpt // 5

    def body(z_hbm, row_hbm, col_hbm, ew_hbm, out_hbm,
             row_v, col_v, ew_v, r0, r1, r2, r3, zb_v, acc_sp,
             g0, g1, g2, g3):
        c = lax.axis_index("c")
        s = lax.axis_index("s")
        rows = [r0, r1, r2, r3]
        gsem = [g0, g1, g2, g3]

        def zb(i, carry):
            for f in range(F // 16):
                zb_v[i, pl.ds(f * 16, 16)] = jnp.zeros((16,), jnp.float32)
            return carry

        lax.fori_loop(0, zch, zb, 0)
        for t in range(5):
            pltpu.sync_copy(zb_v, acc_sp.at[pl.ds(s * rpt + t * zch, zch)])
        plsc.subcore_barrier()

        pltpu.sync_copy(row_hbm.at[s], row_v)
        pltpu.sync_copy(col_hbm.at[s], col_v)
        pltpu.sync_copy(ew_hbm.at[s], ew_v)

        if not fsplit:
            base = c * NHALF

            def fixrow(jj, carry):
                for g in range(K // 16):
                    sl = pl.ds(g * 16, 16)
                    v = col_v[jj, sl] - base
                    ok = (v >= 0) & (v < NHALF)
                    col_v[jj, sl] = jnp.where(ok, v, v & 4095)
                    wv = ew_v[jj, sl]
                    ew_v[jj, sl] = jnp.where(ok, wv, 0.0)
                return carry

            lax.fori_loop(0, nchunk, fixrow, 0)

        def zsrc(j):
            if fsplit:
                return z_hbm.at[c].at[row_v.at[j]]
            return z_hbm.at[row_v.at[j]]

        nfull = nchunk // NBUF
        nrem = nchunk - nfull * NBUF

        def quad(q, carry):
            descs = []
            for t in range(NBUF):
                j = NBUF * q + t
                descs.append(pltpu.async_copy(zsrc(j), rows[t], gsem[t]))
            for t in range(NBUF):
                j = NBUF * q + t
                descs[t].wait()
                _scale_chunk(rows[t], ew_v, j, F)
                pltpu.sync_copy(rows[t], acc_sp.at[col_v.at[j]], add=True)
            return carry

        lax.fori_loop(0, nfull, quad, 0)
        if nrem:
            rdescs = []
            for t in range(nrem):
                j = nfull * NBUF + t
                rdescs.append(pltpu.async_copy(zsrc(j), rows[t], gsem[t]))
            for t in range(nrem):
                j = nfull * NBUF + t
                rdescs[t].wait()
                _scale_chunk(rows[t], ew_v, j, F)
                pltpu.sync_copy(rows[t], acc_sp.at[col_v.at[j]], add=True)
        plsc.subcore_barrier()
        for t in range(5):
            r0w = s * rpt + t * zch
            if fsplit:
                pltpu.sync_copy(acc_sp.at[pl.ds(r0w, zch)],
                                out_hbm.at[c, pl.ds(r0w, zch)])
            else:
                pltpu.sync_copy(acc_sp.at[pl.ds(r0w, zch)],
                                out_hbm.at[pl.ds(c * NHALF + r0w, zch)])

    return body


def _make_agg(F, nchunk, mode):
    fsplit = mode == "fsplit"
    out_sh = (NC, NPAD, F) if fsplit else (NPAD, F)
    acc_rows = NPAD if fsplit else NHALF
    zch = (RPT if fsplit else NHALF // NS) // 5
    return pl.kernel(
        _make_agg_body(F, nchunk, mode),
        out_type=jax.ShapeDtypeStruct(out_sh, jnp.float32),
        mesh=_mesh,
        compiler_params=pltpu.CompilerParams(use_tc_tiling_on_sc=False),
        scratch_types=[
            pltpu.VMEM((nchunk, K), jnp.int32),
            pltpu.VMEM((nchunk, K), jnp.int32),
            pltpu.VMEM((nchunk, K), jnp.float32),
        ] + [pltpu.VMEM((K, F), jnp.float32)] * NBUF + [
            pltpu.VMEM((zch, F), jnp.float32),
            pltpu.VMEM_SHARED((acc_rows, F), jnp.float32),
        ] + [pltpu.SemaphoreType.DMA] * NBUF,
    )


_agg128 = _make_agg(FH, NCHUNK2, "fsplit")
_agg16 = _make_agg(F2, NCHUNK2, "nsplit")


MBLK = 1000


def _tc_a_body(emb, dwT, db, w1aT, w1b, w1c, pn, tl, d0, d1,
               z1_out, z1h_out, dis_out):
    t = jnp.tanh(jnp.dot(emb[...], dwT[...],
                         preferred_element_type=jnp.float32,
                         precision=lax.Precision.HIGHEST) + db[...])
    y = jnp.dot(t, w1aT[...], preferred_element_type=jnp.float32,
                precision=lax.Precision.HIGHEST)
    y = y + pn[...] * w1b[...] + tl[...] * w1c[...]
    deg = d0[...] + d1[...] + 2.0
    dis = jnp.where(deg > 0, lax.rsqrt(deg), 0.0)
    z = dis * y
    z1_out[...] = z
    z1h_out[0, :, :] = z[:, :FH]
    z1h_out[1, :, :] = z[:, FH:]
    dis_out[...] = dis


def _tc_a(emb, dwT, db, w1aT, w1b, w1c, pn, tl, d0, d1):
    return pl.pallas_call(
        _tc_a_body,
        grid=(N // MBLK,),
        in_specs=[
            pl.BlockSpec((MBLK, H), lambda i: (i, 0)),
            pl.BlockSpec((H, H), lambda i: (0, 0)),
            pl.BlockSpec((1, H), lambda i: (0, 0)),
            pl.BlockSpec((H, F1), lambda i: (0, 0)),
            pl.BlockSpec((1, F1), lambda i: (0, 0)),
            pl.BlockSpec((1, F1), lambda i: (0, 0)),
            pl.BlockSpec((MBLK, 1), lambda i: (i, 0)),
            pl.BlockSpec((MBLK, 1), lambda i: (i, 0)),
            pl.BlockSpec((MBLK, 1), lambda i: (i, 0)),
            pl.BlockSpec((MBLK, 1), lambda i: (i, 0)),
        ],
        out_specs=[
            pl.BlockSpec((MBLK, F1), lambda i: (i, 0)),
            pl.BlockSpec((NC, MBLK, FH), lambda i: (0, i, 0)),
            pl.BlockSpec((MBLK, 1), lambda i: (i, 0)),
        ],
        out_shape=[
            jax.ShapeDtypeStruct((N, F1), jnp.float32),
            jax.ShapeDtypeStruct((NC, N, FH), jnp.float32),
            jax.ShapeDtypeStruct((N, 1), jnp.float32),
        ],
    )(emb, dwT, db, w1aT, w1b, w1c, pn, tl, d0, d1)


def _tc_e_body(acc, z1, dis, b1r, w2Tp, z2_out):
    a = jnp.concatenate([acc[0], acc[1]], axis=1)
    h = jnp.maximum(dis[...] * (a + 2.0 * z1[...]) + b1r[...], 0.0)
    y2 = jnp.dot(h, w2Tp[...], preferred_element_type=jnp.float32,
                 precision=lax.Precision.HIGHEST)
    z2_out[...] = dis[...] * y2


def _tc_e(acc, z1, dis, b1r, w2Tp):
    return pl.pallas_call(
        _tc_e_body,
        out_shape=jax.ShapeDtypeStruct((N, F2), jnp.float32),
    )(acc, z1, dis, b1r, w2Tp)


def _tc_g_body(acc, z2, dis, b2p, out):
    out[...] = dis[...] * (acc[...] + 2.0 * z2[...]) + b2p[...]


def _tc_g(acc, z2, dis, b2p):
    return pl.pallas_call(
        _tc_g_body,
        out_shape=jax.ShapeDtypeStruct((N, F2), jnp.float32),
    )(acc, z2, dis, b2p)


def kernel(embedding, p_num, text_len, edge_index, edge_attr,
           dense_w, dense_b, w1, b1, w2, b2):
    row3 = edge_index[0].reshape(NT, NCHUNK, K)
    col3 = edge_index[1].reshape(NT, NCHUNK, K)
    ew3 = edge_attr.reshape(NT, NCHUNK, K)
    row2 = edge_index[0].reshape(NS, NCHUNK2, K)
    col2 = edge_index[1].reshape(NS, NCHUNK2, K)
    ew2 = edge_attr.reshape(NS, NCHUNK2, K)

    deg_parts = _deg(col3, ew3)
    d0 = deg_parts[0, :N][:, None]
    d1 = deg_parts[1, :N][:, None]

    dwT = dense_w.T
    w1aT = w1[:, :H].T
    w1b = w1[:, H][None, :]
    w1c = w1[:, H + 1][None, :]
    z1, z1h, dis = _tc_a(embedding, dwT, dense_b[None, :], w1aT, w1b, w1c,
                         p_num, text_len, d0, d1)

    acc1 = _agg128(z1h, row2, col2, ew2)[:, :N, :]

    w2Tp = jnp.concatenate(
        [w2.T, jnp.zeros((F1, F2 - w2.shape[0]), jnp.float32)], axis=1)
    z2 = _tc_e(acc1, z1, dis, b1[None, :], w2Tp)

    acc2 = _agg16(z2, row2, col2, ew2)[:N, :]

    b2p = jnp.concatenate(
        [b2, jnp.zeros((F2 - b2.shape[0],), jnp.float32)])[None, :]
    out16 = _tc_g(acc2, z2, dis, b2p)
    return out16[:, :10]


# speedup vs baseline: 1.4042x; 1.0863x over previous
import jax
import jax.numpy as jnp
from jax import lax
from jax.experimental import pallas as pl
from jax.experimental.pallas import tpu as pltpu
from jax.experimental.pallas import tpu_sc as plsc

N = 10000
E = 320000
H = 768
F1 = 128
F2 = 16
NC = 2
NS = 16
NT = NC * NS
K = 80
NCHUNK = (E // NT) // K
NCHUNK2 = (E // NS) // K
NPAD = 10240
RPT = NPAD // NS
NHALF = NPAD // 2
FH = F1 // NC
NBUF = 4

_mesh = plsc.VectorSubcoreMesh(core_axis_name="c", subcore_axis_name="s",
                               num_cores=NC)


def _deg_body(col_hbm, ew_hbm, out_hbm, col_v, ew_v, zb_v, deg_sp):
    c = lax.axis_index("c")
    s = lax.axis_index("s")
    zsl = NPAD // NS

    def zb(i, carry):
        zb_v[pl.ds(i * 16, 16)] = jnp.zeros((16,), jnp.float32)
        return carry

    lax.fori_loop(0, zsl // 16, zb, 0)
    pltpu.sync_copy(zb_v, deg_sp.at[pl.ds(s * zsl, zsl)])
    plsc.subcore_barrier()

    wid = c * NS + s
    pltpu.sync_copy(col_hbm.at[wid], col_v)
    pltpu.sync_copy(ew_hbm.at[wid], ew_v)

    def chunk(j, carry):
        pltpu.sync_copy(ew_v.at[j], deg_sp.at[col_v.at[j]], add=True)
        return carry

    lax.fori_loop(0, NCHUNK, chunk, 0)
    plsc.subcore_barrier()
    pltpu.sync_copy(deg_sp.at[pl.ds(s * zsl, zsl)],
                    out_hbm.at[c, pl.ds(s * zsl, zsl)])


_deg = pl.kernel(
    _deg_body,
    out_type=jax.ShapeDtypeStruct((NC, NPAD), jnp.float32),
    mesh=_mesh,
    compiler_params=pltpu.CompilerParams(use_tc_tiling_on_sc=False),
    scratch_types=[
        pltpu.VMEM((NCHUNK, K), jnp.int32),
        pltpu.VMEM((NCHUNK, K), jnp.float32),
        pltpu.VMEM((NPAD // NS,), jnp.float32),
        pltpu.VMEM_SHARED((NPAD,), jnp.float32),
    ],
)


def _scale_chunk(rows_v, ew_v, j, F):
    def grp(g, carry):
        wv = ew_v[j, pl.ds(g * 16, 16)]
        for l in range(16):
            w = wv[l]
            e = g * 16 + l
            for f in range(F // 16):
                sl = pl.ds(f * 16, 16)
                rows_v[e, sl] = rows_v[e, sl] * w
        return carry

    lax.fori_loop(0, K // 16, grp, 0)


def _make_agg_body(F, nchunk, mode):
    nq = nchunk // NBUF
    fsplit = mode == "fsplit"
    rpt = RPT if fsplit else NHALF // NS
    zch = rpt // 5

    def body(z_hbm, row_hbm, col_hbm, ew_hbm, out_hbm,
             row_v, col_v, ew_v, r0, r1, r2, r3, zb_v, acc_sp,
             g0, g1, g2, g3, s0, s1, s2, s3):
        c = lax.axis_index("c")
        s = lax.axis_index("s")
        rows = [r0, r1, r2, r3]
        gsem = [g0, g1, g2, g3]
        ssem = [s0, s1, s2, s3]

        def zb(i, carry):
            for f in range(F // 16):
                zb_v[i, pl.ds(f * 16, 16)] = jnp.zeros((16,), jnp.float32)
            return carry

        lax.fori_loop(0, zch, zb, 0)
        for t in range(5):
            pltpu.sync_copy(zb_v, acc_sp.at[pl.ds(s * rpt + t * zch, zch)])
        plsc.subcore_barrier()

        pltpu.sync_copy(row_hbm.at[s], row_v)
        pltpu.sync_copy(col_hbm.at[s], col_v)
        pltpu.sync_copy(ew_hbm.at[s], ew_v)

        if not fsplit:
            base = c * NHALF

            def fixrow(jj, carry):
                for g in range(K // 16):
                    sl = pl.ds(g * 16, 16)
                    v = col_v[jj, sl] - base
                    ok = (v >= 0) & (v < NHALF)
                    col_v[jj, sl] = jnp.where(ok, v, v & 4095)
                    wv = ew_v[jj, sl]
                    ew_v[jj, sl] = jnp.where(ok, wv, 0.0)
                return carry

            lax.fori_loop(0, nchunk, fixrow, 0)

        def zsrc(j):
            if fsplit:
                return z_hbm.at[c].at[row_v.at[j]]
            return z_hbm.at[row_v.at[j]]

        nfull = nchunk // NBUF
        nrem = nchunk - nfull * NBUF

        def quad(q, carry):
            descs = []
            for t in range(NBUF):
                j = NBUF * q + t
                descs.append(pltpu.async_copy(zsrc(j), rows[t], gsem[t]))
            sdescs = []
            for t in range(NBUF):
                j = NBUF * q + t
                descs[t].wait()
                _scale_chunk(rows[t], ew_v, j, F)
                sdescs.append(pltpu.async_copy(rows[t],
                                               acc_sp.at[col_v.at[j]],
                                               ssem[t], add=True))
            for t in range(NBUF):
                sdescs[t].wait()
            return carry

        lax.fori_loop(0, nfull, quad, 0)
        if nrem:
            rdescs = []
            for t in range(nrem):
                j = nfull * NBUF + t
                rdescs.append(pltpu.async_copy(zsrc(j), rows[t], gsem[t]))
            for t in range(nrem):
                j = nfull * NBUF + t
                rdescs[t].wait()
                _scale_chunk(rows[t], ew_v, j, F)
                pltpu.sync_copy(rows[t], acc_sp.at[col_v.at[j]], add=True)
        plsc.subcore_barrier()
        for t in range(5):
            r0w = s * rpt + t * zch
            if fsplit:
                pltpu.sync_copy(acc_sp.at[pl.ds(r0w, zch)],
                                out_hbm.at[c, pl.ds(r0w, zch)])
            else:
                pltpu.sync_copy(acc_sp.at[pl.ds(r0w, zch)],
                                out_hbm.at[pl.ds(c * NHALF + r0w, zch)])

    return body


def _make_agg(F, nchunk, mode):
    fsplit = mode == "fsplit"
    out_sh = (NC, NPAD, F) if fsplit else (NPAD, F)
    acc_rows = NPAD if fsplit else NHALF
    zch = (RPT if fsplit else NHALF // NS) // 5
    return pl.kernel(
        _make_agg_body(F, nchunk, mode),
        out_type=jax.ShapeDtypeStruct(out_sh, jnp.float32),
        mesh=_mesh,
        compiler_params=pltpu.CompilerParams(use_tc_tiling_on_sc=False),
        scratch_types=[
            pltpu.VMEM((nchunk, K), jnp.int32),
            pltpu.VMEM((nchunk, K), jnp.int32),
            pltpu.VMEM((nchunk, K), jnp.float32),
        ] + [pltpu.VMEM((K, F), jnp.float32)] * NBUF + [
            pltpu.VMEM((zch, F), jnp.float32),
            pltpu.VMEM_SHARED((acc_rows, F), jnp.float32),
        ] + [pltpu.SemaphoreType.DMA] * (2 * NBUF),
    )


_agg128 = _make_agg(FH, NCHUNK2, "fsplit")
_agg16 = _make_agg(F2, NCHUNK2, "nsplit")


MBLK = 1000


def _tc_a_body(emb, dwT, db, w1aT, w1b, w1c, pn, tl, d0, d1,
               z1_out, z1h_out, dis_out):
    t = jnp.tanh(jnp.dot(emb[...], dwT[...],
                         preferred_element_type=jnp.float32,
                         precision=lax.Precision.HIGHEST) + db[...])
    y = jnp.dot(t, w1aT[...], preferred_element_type=jnp.float32,
                precision=lax.Precision.HIGHEST)
    y = y + pn[...] * w1b[...] + tl[...] * w1c[...]
    deg = d0[...] + d1[...] + 2.0
    dis = jnp.where(deg > 0, lax.rsqrt(deg), 0.0)
    z = dis * y
    z1_out[...] = z
    z1h_out[0, :, :] = z[:, :FH]
    z1h_out[1, :, :] = z[:, FH:]
    dis_out[...] = dis


def _tc_a(emb, dwT, db, w1aT, w1b, w1c, pn, tl, d0, d1):
    return pl.pallas_call(
        _tc_a_body,
        grid=(N // MBLK,),
        in_specs=[
            pl.BlockSpec((MBLK, H), lambda i: (i, 0)),
            pl.BlockSpec((H, H), lambda i: (0, 0)),
            pl.BlockSpec((1, H), lambda i: (0, 0)),
            pl.BlockSpec((H, F1), lambda i: (0, 0)),
            pl.BlockSpec((1, F1), lambda i: (0, 0)),
            pl.BlockSpec((1, F1), lambda i: (0, 0)),
            pl.BlockSpec((MBLK, 1), lambda i: (i, 0)),
            pl.BlockSpec((MBLK, 1), lambda i: (i, 0)),
            pl.BlockSpec((MBLK, 1), lambda i: (i, 0)),
            pl.BlockSpec((MBLK, 1), lambda i: (i, 0)),
        ],
        out_specs=[
            pl.BlockSpec((MBLK, F1), lambda i: (i, 0)),
            pl.BlockSpec((NC, MBLK, FH), lambda i: (0, i, 0)),
            pl.BlockSpec((MBLK, 1), lambda i: (i, 0)),
        ],
        out_shape=[
            jax.ShapeDtypeStruct((N, F1), jnp.float32),
            jax.ShapeDtypeStruct((NC, N, FH), jnp.float32),
            jax.ShapeDtypeStruct((N, 1), jnp.float32),
        ],
    )(emb, dwT, db, w1aT, w1b, w1c, pn, tl, d0, d1)


def _tc_e_body(acc, z1, dis, b1r, w2Tp, z2_out):
    a = jnp.concatenate([acc[0], acc[1]], axis=1)
    h = jnp.maximum(dis[...] * (a + 2.0 * z1[...]) + b1r[...], 0.0)
    y2 = jnp.dot(h, w2Tp[...], preferred_element_type=jnp.float32,
                 precision=lax.Precision.HIGHEST)
    z2_out[...] = dis[...] * y2


def _tc_e(acc, z1, dis, b1r, w2Tp):
    return pl.pallas_call(
        _tc_e_body,
        out_shape=jax.ShapeDtypeStruct((N, F2), jnp.float32),
    )(acc, z1, dis, b1r, w2Tp)


def _tc_g_body(acc, z2, dis, b2p, out):
    out[...] = dis[...] * (acc[...] + 2.0 * z2[...]) + b2p[...]


def _tc_g(acc, z2, dis, b2p):
    return pl.pallas_call(
        _tc_g_body,
        out_shape=jax.ShapeDtypeStruct((N, F2), jnp.float32),
    )(acc, z2, dis, b2p)


def kernel(embedding, p_num, text_len, edge_index, edge_attr,
           dense_w, dense_b, w1, b1, w2, b2):
    row3 = edge_index[0].reshape(NT, NCHUNK, K)
    col3 = edge_index[1].reshape(NT, NCHUNK, K)
    ew3 = edge_attr.reshape(NT, NCHUNK, K)
    row2 = edge_index[0].reshape(NS, NCHUNK2, K)
    col2 = edge_index[1].reshape(NS, NCHUNK2, K)
    ew2 = edge_attr.reshape(NS, NCHUNK2, K)

    deg_parts = _deg(col3, ew3)
    d0 = deg_parts[0, :N][:, None]
    d1 = deg_parts[1, :N][:, None]

    dwT = dense_w.T
    w1aT = w1[:, :H].T
    w1b = w1[:, H][None, :]
    w1c = w1[:, H + 1][None, :]
    z1, z1h, dis = _tc_a(embedding, dwT, dense_b[None, :], w1aT, w1b, w1c,
                         p_num, text_len, d0, d1)

    acc1 = _agg128(z1h, row2, col2, ew2)[:, :N, :]

    w2Tp = jnp.concatenate(
        [w2.T, jnp.zeros((F1, F2 - w2.shape[0]), jnp.float32)], axis=1)
    z2 = _tc_e(acc1, z1, dis, b1[None, :], w2Tp)

    acc2 = _agg16(z2, row2, col2, ew2)[:N, :]

    b2p = jnp.concatenate(
        [b2, jnp.zeros((F2 - b2.shape[0],), jnp.float32)])[None, :]
    out16 = _tc_g(acc2, z2, dis, b2p)
    return out16[:, :10]


# speedup vs baseline: 2.1457x; 1.5280x over previous
import jax
import jax.numpy as jnp
from jax import lax
from jax.experimental import pallas as pl
from jax.experimental.pallas import tpu as pltpu
from jax.experimental.pallas import tpu_sc as plsc

N = 10000
E = 320000
H = 768
F1 = 128
F2 = 16
NC = 2
NS = 16
NT = NC * NS
K = 80
NCHUNK = (E // NT) // K
NCHUNK2 = (E // NS) // K
NPAD = 10240
RPT = NPAD // NS
NHALF = NPAD // 2
FH = F1 // NC
NBUF = 4

_mesh = plsc.VectorSubcoreMesh(core_axis_name="c", subcore_axis_name="s",
                               num_cores=NC)


def _deg_body(col_hbm, ew_hbm, out_hbm, col_v, ew_v, zb_v, deg_sp):
    c = lax.axis_index("c")
    s = lax.axis_index("s")
    zsl = NPAD // NS

    def zb(i, carry):
        zb_v[pl.ds(i * 16, 16)] = jnp.zeros((16,), jnp.float32)
        return carry

    lax.fori_loop(0, zsl // 16, zb, 0)
    pltpu.sync_copy(zb_v, deg_sp.at[pl.ds(s * zsl, zsl)])
    plsc.subcore_barrier()

    wid = c * NS + s
    pltpu.sync_copy(col_hbm.at[wid], col_v)
    pltpu.sync_copy(ew_hbm.at[wid], ew_v)

    def chunk(j, carry):
        pltpu.sync_copy(ew_v.at[j], deg_sp.at[col_v.at[j]], add=True)
        return carry

    lax.fori_loop(0, NCHUNK, chunk, 0)
    plsc.subcore_barrier()
    pltpu.sync_copy(deg_sp.at[pl.ds(s * zsl, zsl)],
                    out_hbm.at[c, pl.ds(s * zsl, zsl)])


_deg = pl.kernel(
    _deg_body,
    out_type=jax.ShapeDtypeStruct((NC, NPAD), jnp.float32),
    mesh=_mesh,
    compiler_params=pltpu.CompilerParams(use_tc_tiling_on_sc=False),
    scratch_types=[
        pltpu.VMEM((NCHUNK, K), jnp.int32),
        pltpu.VMEM((NCHUNK, K), jnp.float32),
        pltpu.VMEM((NPAD // NS,), jnp.float32),
        pltpu.VMEM_SHARED((NPAD,), jnp.float32),
    ],
)


def _scale_chunk(rows_v, ew_v, j, F):
    @plsc.parallel_loop(0, K // 16, unroll=2)
    def grp(g):
        wv = ew_v[j, pl.ds(g * 16, 16)]
        for l in range(16):
            w = wv[l]
            e = g * 16 + l
            for f in range(F // 16):
                sl = pl.ds(f * 16, 16)
                rows_v[e, sl] = rows_v[e, sl] * w


def _make_agg_body(F, nchunk, mode):
    nq = nchunk // NBUF
    fsplit = mode == "fsplit"
    rpt = RPT if fsplit else NHALF // NS
    zch = rpt // 5

    def body(z_hbm, row_hbm, col_hbm, ew_hbm, out_hbm,
             row_v, col_v, ew_v, r0, r1, r2, r3, zb_v, acc_sp,
             g0, g1, g2, g3, s0, s1, s2, s3):
        c = lax.axis_index("c")
        s = lax.axis_index("s")
        rows = [r0, r1, r2, r3]
        gsem = [g0, g1, g2, g3]
        ssem = [s0, s1, s2, s3]

        def zb(i, carry):
            for f in range(F // 16):
                zb_v[i, pl.ds(f * 16, 16)] = jnp.zeros((16,), jnp.float32)
            return carry

        lax.fori_loop(0, zch, zb, 0)
        for t in range(5):
            pltpu.sync_copy(zb_v, acc_sp.at[pl.ds(s * rpt + t * zch, zch)])
        plsc.subcore_barrier()

        pltpu.sync_copy(row_hbm.at[s], row_v)
        pltpu.sync_copy(col_hbm.at[s], col_v)
        pltpu.sync_copy(ew_hbm.at[s], ew_v)

        if not fsplit:
            base = c * NHALF

            def fixrow(jj, carry):
                for g in range(K // 16):
                    sl = pl.ds(g * 16, 16)
                    v = col_v[jj, sl] - base
                    ok = (v >= 0) & (v < NHALF)
                    col_v[jj, sl] = jnp.where(ok, v, v & 4095)
                    wv = ew_v[jj, sl]
                    ew_v[jj, sl] = jnp.where(ok, wv, 0.0)
                return carry

            lax.fori_loop(0, nchunk, fixrow, 0)

        def zsrc(j):
            if fsplit:
                return z_hbm.at[c].at[row_v.at[j]]
            return z_hbm.at[row_v.at[j]]

        nfull = nchunk // NBUF
        nrem = nchunk - nfull * NBUF

        def quad(q, carry):
            descs = []
            for t in range(NBUF):
                j = NBUF * q + t
                descs.append(pltpu.async_copy(zsrc(j), rows[t], gsem[t]))
            sdescs = []
            for t in range(NBUF):
                j = NBUF * q + t
                descs[t].wait()
                _scale_chunk(rows[t], ew_v, j, F)
                sdescs.append(pltpu.async_copy(rows[t],
                                               acc_sp.at[col_v.at[j]],
                                               ssem[t], add=True))
            for t in range(NBUF):
                sdescs[t].wait()
            return carry

        lax.fori_loop(0, nfull, quad, 0)
        if nrem:
            rdescs = []
            for t in range(nrem):
                j = nfull * NBUF + t
                rdescs.append(pltpu.async_copy(zsrc(j), rows[t], gsem[t]))
            for t in range(nrem):
                j = nfull * NBUF + t
                rdescs[t].wait()
                _scale_chunk(rows[t], ew_v, j, F)
                pltpu.sync_copy(rows[t], acc_sp.at[col_v.at[j]], add=True)
        plsc.subcore_barrier()
        for t in range(5):
            r0w = s * rpt + t * zch
            if fsplit:
                pltpu.sync_copy(acc_sp.at[pl.ds(r0w, zch)],
                                out_hbm.at[c, pl.ds(r0w, zch)])
            else:
                pltpu.sync_copy(acc_sp.at[pl.ds(r0w, zch)],
                                out_hbm.at[pl.ds(c * NHALF + r0w, zch)])

    return body


def _make_agg(F, nchunk, mode):
    fsplit = mode == "fsplit"
    out_sh = (NC, NPAD, F) if fsplit else (NPAD, F)
    acc_rows = NPAD if fsplit else NHALF
    zch = (RPT if fsplit else NHALF // NS) // 5
    return pl.kernel(
        _make_agg_body(F, nchunk, mode),
        out_type=jax.ShapeDtypeStruct(out_sh, jnp.float32),
        mesh=_mesh,
        compiler_params=pltpu.CompilerParams(use_tc_tiling_on_sc=False),
        scratch_types=[
            pltpu.VMEM((nchunk, K), jnp.int32),
            pltpu.VMEM((nchunk, K), jnp.int32),
            pltpu.VMEM((nchunk, K), jnp.float32),
        ] + [pltpu.VMEM((K, F), jnp.float32)] * NBUF + [
            pltpu.VMEM((zch, F), jnp.float32),
            pltpu.VMEM_SHARED((acc_rows, F), jnp.float32),
        ] + [pltpu.SemaphoreType.DMA] * (2 * NBUF),
    )


_agg128 = _make_agg(FH, NCHUNK2, "fsplit")
_agg16 = _make_agg(F2, NCHUNK2, "nsplit")


MBLK = 1000


def _tc_a_body(emb, dwT, db, w1aT, w1b, w1c, pn, tl, d0, d1,
               z1_out, z1h_out, dis_out):
    t = jnp.tanh(jnp.dot(emb[...], dwT[...],
                         preferred_element_type=jnp.float32,
                         precision=lax.Precision.HIGHEST) + db[...])
    y = jnp.dot(t, w1aT[...], preferred_element_type=jnp.float32,
                precision=lax.Precision.HIGHEST)
    y = y + pn[...] * w1b[...] + tl[...] * w1c[...]
    deg = d0[...] + d1[...] + 2.0
    dis = jnp.where(deg > 0, lax.rsqrt(deg), 0.0)
    z = dis * y
    z1_out[...] = z
    z1h_out[0, :, :] = z[:, :FH]
    z1h_out[1, :, :] = z[:, FH:]
    dis_out[...] = dis


def _tc_a(emb, dwT, db, w1aT, w1b, w1c, pn, tl, d0, d1):
    return pl.pallas_call(
        _tc_a_body,
        grid=(N // MBLK,),
        in_specs=[
            pl.BlockSpec((MBLK, H), lambda i: (i, 0)),
            pl.BlockSpec((H, H), lambda i: (0, 0)),
            pl.BlockSpec((1, H), lambda i: (0, 0)),
            pl.BlockSpec((H, F1), lambda i: (0, 0)),
            pl.BlockSpec((1, F1), lambda i: (0, 0)),
            pl.BlockSpec((1, F1), lambda i: (0, 0)),
            pl.BlockSpec((MBLK, 1), lambda i: (i, 0)),
            pl.BlockSpec((MBLK, 1), lambda i: (i, 0)),
            pl.BlockSpec((MBLK, 1), lambda i: (i, 0)),
            pl.BlockSpec((MBLK, 1), lambda i: (i, 0)),
        ],
        out_specs=[
            pl.BlockSpec((MBLK, F1), lambda i: (i, 0)),
            pl.BlockSpec((NC, MBLK, FH), lambda i: (0, i, 0)),
            pl.BlockSpec((MBLK, 1), lambda i: (i, 0)),
        ],
        out_shape=[
            jax.ShapeDtypeStruct((N, F1), jnp.float32),
            jax.ShapeDtypeStruct((NC, N, FH), jnp.float32),
            jax.ShapeDtypeStruct((N, 1), jnp.float32),
        ],
    )(emb, dwT, db, w1aT, w1b, w1c, pn, tl, d0, d1)


def _tc_e_body(acc, z1, dis, b1r, w2Tp, z2_out):
    a = jnp.concatenate([acc[0], acc[1]], axis=1)
    h = jnp.maximum(dis[...] * (a + 2.0 * z1[...]) + b1r[...], 0.0)
    y2 = jnp.dot(h, w2Tp[...], preferred_element_type=jnp.float32,
                 precision=lax.Precision.HIGHEST)
    z2_out[...] = dis[...] * y2


def _tc_e(acc, z1, dis, b1r, w2Tp):
    return pl.pallas_call(
        _tc_e_body,
        out_shape=jax.ShapeDtypeStruct((N, F2), jnp.float32),
    )(acc, z1, dis, b1r, w2Tp)


def _tc_g_body(acc, z2, dis, b2p, out):
    out[...] = dis[...] * (acc[...] + 2.0 * z2[...]) + b2p[...]


def _tc_g(acc, z2, dis, b2p):
    return pl.pallas_call(
        _tc_g_body,
        out_shape=jax.ShapeDtypeStruct((N, F2), jnp.float32),
    )(acc, z2, dis, b2p)


def kernel(embedding, p_num, text_len, edge_index, edge_attr,
           dense_w, dense_b, w1, b1, w2, b2):
    row3 = edge_index[0].reshape(NT, NCHUNK, K)
    col3 = edge_index[1].reshape(NT, NCHUNK, K)
    ew3 = edge_attr.reshape(NT, NCHUNK, K)
    row2 = edge_index[0].reshape(NS, NCHUNK2, K)
    col2 = edge_index[1].reshape(NS, NCHUNK2, K)
    ew2 = edge_attr.reshape(NS, NCHUNK2, K)

    deg_parts = _deg(col3, ew3)
    d0 = deg_parts[0, :N][:, None]
    d1 = deg_parts[1, :N][:, None]

    dwT = dense_w.T
    w1aT = w1[:, :H].T
    w1b = w1[:, H][None, :]
    w1c = w1[:, H + 1][None, :]
    z1, z1h, dis = _tc_a(embedding, dwT, dense_b[None, :], w1aT, w1b, w1c,
                         p_num, text_len, d0, d1)

    acc1 = _agg128(z1h, row2, col2, ew2)[:, :N, :]

    w2Tp = jnp.concatenate(
        [w2.T, jnp.zeros((F1, F2 - w2.shape[0]), jnp.float32)], axis=1)
    z2 = _tc_e(acc1, z1, dis, b1[None, :], w2Tp)

    acc2 = _agg16(z2, row2, col2, ew2)[:N, :]

    b2p = jnp.concatenate(
        [b2, jnp.zeros((F2 - b2.shape[0],), jnp.float32)])[None, :]
    out16 = _tc_g(acc2, z2, dis, b2p)
    return out16[:, :10]


# speedup vs baseline: 2.7798x; 1.2955x over previous
import jax
import jax.numpy as jnp
from jax import lax
from jax.experimental import pallas as pl
from jax.experimental.pallas import tpu as pltpu
from jax.experimental.pallas import tpu_sc as plsc

N = 10000
E = 320000
H = 768
F1 = 128
F2 = 16
NC = 2
NS = 16
NT = NC * NS
K = 80
NCHUNK = (E // NT) // K
NCHUNK2 = (E // NS) // K
NPAD = 10240
RPT = NPAD // NS
NHALF = NPAD // 2
FH = F1 // NC

_mesh = plsc.VectorSubcoreMesh(core_axis_name="c", subcore_axis_name="s",
                               num_cores=NC)


def _deg_body(col_hbm, ew_hbm, out_hbm, col_v, ew_v, zb_v, deg_sp):
    c = lax.axis_index("c")
    s = lax.axis_index("s")
    zsl = NPAD // NS

    def zb(i, carry):
        zb_v[pl.ds(i * 16, 16)] = jnp.zeros((16,), jnp.float32)
        return carry

    lax.fori_loop(0, zsl // 16, zb, 0)
    pltpu.sync_copy(zb_v, deg_sp.at[pl.ds(s * zsl, zsl)])
    plsc.subcore_barrier()

    wid = c * NS + s
    pltpu.sync_copy(col_hbm.at[wid], col_v)
    pltpu.sync_copy(ew_hbm.at[wid], ew_v)

    def chunk(j, carry):
        pltpu.sync_copy(ew_v.at[j], deg_sp.at[col_v.at[j]], add=True)
        return carry

    lax.fori_loop(0, NCHUNK, chunk, 0)
    plsc.subcore_barrier()
    pltpu.sync_copy(deg_sp.at[pl.ds(s * zsl, zsl)],
                    out_hbm.at[c, pl.ds(s * zsl, zsl)])


_deg = pl.kernel(
    _deg_body,
    out_type=jax.ShapeDtypeStruct((NC, NPAD), jnp.float32),
    mesh=_mesh,
    compiler_params=pltpu.CompilerParams(use_tc_tiling_on_sc=False),
    scratch_types=[
        pltpu.VMEM((NCHUNK, K), jnp.int32),
        pltpu.VMEM((NCHUNK, K), jnp.float32),
        pltpu.VMEM((NPAD // NS,), jnp.float32),
        pltpu.VMEM_SHARED((NPAD,), jnp.float32),
    ],
)


def _scale_chunk(rows_v, ew_v, j, F):
    @plsc.parallel_loop(0, K // 16, unroll=2)
    def grp(g):
        wv = ew_v[j, pl.ds(g * 16, 16)]
        for l in range(16):
            w = wv[l]
            e = g * 16 + l
            for f in range(F // 16):
                sl = pl.ds(f * 16, 16)
                rows_v[e, sl] = rows_v[e, sl] * w


def _make_agg_body(F, nchunk, mode, nbuf):
    fsplit = mode == "fsplit"
    rpt = RPT if fsplit else NHALF // NS
    zch = rpt // 5

    def body(z_hbm, row_hbm, col_hbm, ew_hbm, out_hbm, *scratch):
        row_v, col_v, ew_v = scratch[0:3]
        rows = list(scratch[3:3 + nbuf])
        zb_v, acc_sp = scratch[3 + nbuf:5 + nbuf]
        gsem = list(scratch[5 + nbuf:5 + 2 * nbuf])
        ssem = list(scratch[5 + 2 * nbuf:5 + 3 * nbuf])
        c = lax.axis_index("c")
        s = lax.axis_index("s")

        def zb(i, carry):
            for f in range(F // 16):
                zb_v[i, pl.ds(f * 16, 16)] = jnp.zeros((16,), jnp.float32)
            return carry

        lax.fori_loop(0, zch, zb, 0)
        for t in range(5):
            pltpu.sync_copy(zb_v, acc_sp.at[pl.ds(s * rpt + t * zch, zch)])
        plsc.subcore_barrier()

        pltpu.sync_copy(row_hbm.at[s], row_v)
        pltpu.sync_copy(col_hbm.at[s], col_v)
        pltpu.sync_copy(ew_hbm.at[s], ew_v)

        if not fsplit:
            base = c * NHALF

            def fixrow(jj, carry):
                for g in range(K // 16):
                    sl = pl.ds(g * 16, 16)
                    v = col_v[jj, sl] - base
                    ok = (v >= 0) & (v < NHALF)
                    col_v[jj, sl] = jnp.where(ok, v, v & 4095)
                    wv = ew_v[jj, sl]
                    ew_v[jj, sl] = jnp.where(ok, wv, 0.0)
                return carry

            lax.fori_loop(0, nchunk, fixrow, 0)

        def zsrc(j):
            if fsplit:
                return z_hbm.at[c].at[row_v.at[j]]
            return z_hbm.at[row_v.at[j]]

        nfull = nchunk // nbuf
        nrem = nchunk - nfull * nbuf

        def quad(q, carry):
            descs = []
            for t in range(nbuf):
                j = nbuf * q + t
                descs.append(pltpu.async_copy(zsrc(j), rows[t], gsem[t]))
            sdescs = []
            for t in range(nbuf):
                j = nbuf * q + t
                descs[t].wait()
                _scale_chunk(rows[t], ew_v, j, F)
                sdescs.append(pltpu.async_copy(rows[t],
                                               acc_sp.at[col_v.at[j]],
                                               ssem[t], add=True))
            for t in range(nbuf):
                sdescs[t].wait()
            return carry

        lax.fori_loop(0, nfull, quad, 0)
        if nrem:
            rdescs = []
            for t in range(nrem):
                j = nfull * nbuf + t
                rdescs.append(pltpu.async_copy(zsrc(j), rows[t], gsem[t]))
            for t in range(nrem):
                j = nfull * nbuf + t
                rdescs[t].wait()
                _scale_chunk(rows[t], ew_v, j, F)
                pltpu.sync_copy(rows[t], acc_sp.at[col_v.at[j]], add=True)
        plsc.subcore_barrier()
        for t in range(5):
            r0w = s * rpt + t * zch
            if fsplit:
                pltpu.sync_copy(acc_sp.at[pl.ds(r0w, zch)],
                                out_hbm.at[c, pl.ds(r0w, zch)])
            else:
                pltpu.sync_copy(acc_sp.at[pl.ds(r0w, zch)],
                                out_hbm.at[pl.ds(c * NHALF + r0w, zch)])

    return body


def _make_agg(F, nchunk, mode, nbuf):
    fsplit = mode == "fsplit"
    out_sh = (NC, NPAD, F) if fsplit else (NPAD, F)
    acc_rows = NPAD if fsplit else NHALF
    zch = (RPT if fsplit else NHALF // NS) // 5
    return pl.kernel(
        _make_agg_body(F, nchunk, mode, nbuf),
        out_type=jax.ShapeDtypeStruct(out_sh, jnp.float32),
        mesh=_mesh,
        compiler_params=pltpu.CompilerParams(use_tc_tiling_on_sc=False),
        scratch_types=[
            pltpu.VMEM((nchunk, K), jnp.int32),
            pltpu.VMEM((nchunk, K), jnp.int32),
            pltpu.VMEM((nchunk, K), jnp.float32),
        ] + [pltpu.VMEM((K, F), jnp.float32)] * nbuf + [
            pltpu.VMEM((zch, F), jnp.float32),
            pltpu.VMEM_SHARED((acc_rows, F), jnp.float32),
        ] + [pltpu.SemaphoreType.DMA] * (2 * nbuf),
    )


_agg128 = _make_agg(FH, NCHUNK2, "fsplit", 4)
_agg16 = _make_agg(F2, NCHUNK2, "nsplit", 8)


MBLK = 1000


def _tc_a_body(emb, dwT, db, w1aT, w1b, w1c, pn, tl, d0, d1,
               z1_out, z1h_out, dis_out):
    t = jnp.tanh(jnp.dot(emb[...], dwT[...],
                         preferred_element_type=jnp.float32) + db[...])
    y = jnp.dot(t, w1aT[...], preferred_element_type=jnp.float32)
    y = y + pn[...] * w1b[...] + tl[...] * w1c[...]
    deg = d0[...] + d1[...] + 2.0
    dis = jnp.where(deg > 0, lax.rsqrt(deg), 0.0)
    z = dis * y
    z1_out[...] = z
    z1h_out[0, :, :] = z[:, :FH]
    z1h_out[1, :, :] = z[:, FH:]
    dis_out[...] = dis


def _tc_a(emb, dwT, db, w1aT, w1b, w1c, pn, tl, d0, d1):
    return pl.pallas_call(
        _tc_a_body,
        grid=(N // MBLK,),
        in_specs=[
            pl.BlockSpec((MBLK, H), lambda i: (i, 0)),
            pl.BlockSpec((H, H), lambda i: (0, 0)),
            pl.BlockSpec((1, H), lambda i: (0, 0)),
            pl.BlockSpec((H, F1), lambda i: (0, 0)),
            pl.BlockSpec((1, F1), lambda i: (0, 0)),
            pl.BlockSpec((1, F1), lambda i: (0, 0)),
            pl.BlockSpec((MBLK, 1), lambda i: (i, 0)),
            pl.BlockSpec((MBLK, 1), lambda i: (i, 0)),
            pl.BlockSpec((MBLK, 1), lambda i: (i, 0)),
            pl.BlockSpec((MBLK, 1), lambda i: (i, 0)),
        ],
        out_specs=[
            pl.BlockSpec((MBLK, F1), lambda i: (i, 0)),
            pl.BlockSpec((NC, MBLK, FH), lambda i: (0, i, 0)),
            pl.BlockSpec((MBLK, 1), lambda i: (i, 0)),
        ],
        out_shape=[
            jax.ShapeDtypeStruct((N, F1), jnp.float32),
            jax.ShapeDtypeStruct((NC, N, FH), jnp.float32),
            jax.ShapeDtypeStruct((N, 1), jnp.float32),
        ],
    )(emb, dwT, db, w1aT, w1b, w1c, pn, tl, d0, d1)


def _tc_e_body(acc, z1, dis, b1r, w2Tp, z2_out):
    a = jnp.concatenate([acc[0], acc[1]], axis=1)
    h = jnp.maximum(dis[...] * (a + 2.0 * z1[...]) + b1r[...], 0.0)
    y2 = jnp.dot(h, w2Tp[...], preferred_element_type=jnp.float32,
                 precision=lax.Precision.HIGHEST)
    z2_out[...] = dis[...] * y2


def _tc_e(acc, z1, dis, b1r, w2Tp):
    return pl.pallas_call(
        _tc_e_body,
        out_shape=jax.ShapeDtypeStruct((N, F2), jnp.float32),
    )(acc, z1, dis, b1r, w2Tp)


def _tc_g_body(acc, z2, dis, b2p, out):
    out[...] = dis[...] * (acc[...] + 2.0 * z2[...]) + b2p[...]


def _tc_g(acc, z2, dis, b2p):
    return pl.pallas_call(
        _tc_g_body,
        out_shape=jax.ShapeDtypeStruct((N, F2), jnp.float32),
    )(acc, z2, dis, b2p)


def kernel(embedding, p_num, text_len, edge_index, edge_attr,
           dense_w, dense_b, w1, b1, w2, b2):
    row3 = edge_index[0].reshape(NT, NCHUNK, K)
    col3 = edge_index[1].reshape(NT, NCHUNK, K)
    ew3 = edge_attr.reshape(NT, NCHUNK, K)
    row2 = edge_index[0].reshape(NS, NCHUNK2, K)
    col2 = edge_index[1].reshape(NS, NCHUNK2, K)
    ew2 = edge_attr.reshape(NS, NCHUNK2, K)

    deg_parts = _deg(col3, ew3)
    d0 = deg_parts[0, :N][:, None]
    d1 = deg_parts[1, :N][:, None]

    dwT = dense_w.T
    w1aT = w1[:, :H].T
    w1b = w1[:, H][None, :]
    w1c = w1[:, H + 1][None, :]
    z1, z1h, dis = _tc_a(embedding, dwT, dense_b[None, :], w1aT, w1b, w1c,
                         p_num, text_len, d0, d1)

    acc1 = _agg128(z1h, row2, col2, ew2)[:, :N, :]

    w2Tp = jnp.concatenate(
        [w2.T, jnp.zeros((F1, F2 - w2.shape[0]), jnp.float32)], axis=1)
    z2 = _tc_e(acc1, z1, dis, b1[None, :], w2Tp)

    acc2 = _agg16(z2, row2, col2, ew2)[:N, :]

    b2p = jnp.concatenate(
        [b2, jnp.zeros((F2 - b2.shape[0],), jnp.float32)])[None, :]
    out16 = _tc_g(acc2, z2, dis, b2p)
    return out16[:, :10]


# speedup vs baseline: 2.7880x; 1.0029x over previous
import jax
import jax.numpy as jnp
from jax import lax
from jax.experimental import pallas as pl
from jax.experimental.pallas import tpu as pltpu
from jax.experimental.pallas import tpu_sc as plsc

N = 10000
E = 320000
H = 768
F1 = 128
F2 = 16
NC = 2
NS = 16
NT = NC * NS
K = 80
NCHUNK = (E // NT) // K
NCHUNK2 = (E // NS) // K
NPAD = 10240
RPT = NPAD // NS
NHALF = NPAD // 2
FH = F1 // NC

_mesh = plsc.VectorSubcoreMesh(core_axis_name="c", subcore_axis_name="s",
                               num_cores=NC)


def _deg_body(col_hbm, ew_hbm, out_hbm, col_v, ew_v, zb_v, deg_sp):
    c = lax.axis_index("c")
    s = lax.axis_index("s")
    zsl = NPAD // NS

    def zb(i, carry):
        zb_v[pl.ds(i * 16, 16)] = jnp.zeros((16,), jnp.float32)
        return carry

    lax.fori_loop(0, zsl // 16, zb, 0)
    pltpu.sync_copy(zb_v, deg_sp.at[pl.ds(s * zsl, zsl)])
    plsc.subcore_barrier()

    wid = c * NS + s
    pltpu.sync_copy(col_hbm.at[wid], col_v)
    pltpu.sync_copy(ew_hbm.at[wid], ew_v)

    def chunk(j, carry):
        pltpu.sync_copy(ew_v.at[j], deg_sp.at[col_v.at[j]], add=True)
        return carry

    lax.fori_loop(0, NCHUNK, chunk, 0)
    plsc.subcore_barrier()
    pltpu.sync_copy(deg_sp.at[pl.ds(s * zsl, zsl)],
                    out_hbm.at[c, pl.ds(s * zsl, zsl)])


_deg = pl.kernel(
    _deg_body,
    out_type=jax.ShapeDtypeStruct((NC, NPAD), jnp.float32),
    mesh=_mesh,
    compiler_params=pltpu.CompilerParams(use_tc_tiling_on_sc=False),
    scratch_types=[
        pltpu.VMEM((NCHUNK, K), jnp.int32),
        pltpu.VMEM((NCHUNK, K), jnp.float32),
        pltpu.VMEM((NPAD // NS,), jnp.float32),
        pltpu.VMEM_SHARED((NPAD,), jnp.float32),
    ],
)


def _scale_chunk(rows_v, ew_v, j, F):
    @plsc.parallel_loop(0, K // 16, unroll=2)
    def grp(g):
        wv = ew_v[j, pl.ds(g * 16, 16)]
        for l in range(16):
            w = wv[l]
            e = g * 16 + l
            for f in range(F // 16):
                sl = pl.ds(f * 16, 16)
                rows_v[e, sl] = rows_v[e, sl] * w


def _make_agg_body(F, nchunk, mode, nbuf):
    fsplit = mode == "fsplit"
    rpt = RPT if fsplit else NHALF // NS
    zch = rpt // 5

    def body(z_hbm, row_hbm, col_hbm, ew_hbm, out_hbm, *scratch):
        row_v, col_v, ew_v = scratch[0:3]
        rows = list(scratch[3:3 + nbuf])
        zb_v, acc_sp = scratch[3 + nbuf:5 + nbuf]
        gsem = list(scratch[5 + nbuf:5 + 2 * nbuf])
        ssem = list(scratch[5 + 2 * nbuf:5 + 3 * nbuf])
        c = lax.axis_index("c")
        s = lax.axis_index("s")

        def zb(i, carry):
            for f in range(F // 16):
                zb_v[i, pl.ds(f * 16, 16)] = jnp.zeros((16,), jnp.float32)
            return carry

        lax.fori_loop(0, zch, zb, 0)
        for t in range(5):
            pltpu.sync_copy(zb_v, acc_sp.at[pl.ds(s * rpt + t * zch, zch)])
        plsc.subcore_barrier()

        pltpu.sync_copy(row_hbm.at[s], row_v)
        pltpu.sync_copy(col_hbm.at[s], col_v)
        pltpu.sync_copy(ew_hbm.at[s], ew_v)

        if not fsplit:
            base = c * NHALF

            def fixrow(jj, carry):
                for g in range(K // 16):
                    sl = pl.ds(g * 16, 16)
                    v = col_v[jj, sl] - base
                    ok = (v >= 0) & (v < NHALF)
                    col_v[jj, sl] = jnp.where(ok, v, v & 4095)
                    wv = ew_v[jj, sl]
                    ew_v[jj, sl] = jnp.where(ok, wv, 0.0)
                return carry

            lax.fori_loop(0, nchunk, fixrow, 0)

        def zsrc(j):
            if fsplit:
                return z_hbm.at[c].at[row_v.at[j]]
            return z_hbm.at[row_v.at[j]]

        nfull = nchunk // nbuf
        nrem = nchunk - nfull * nbuf

        def quad(q, carry):
            descs = []
            for t in range(nbuf):
                j = nbuf * q + t
                descs.append(pltpu.async_copy(zsrc(j), rows[t], gsem[t]))
            sdescs = []
            for t in range(nbuf):
                j = nbuf * q + t
                descs[t].wait()
                _scale_chunk(rows[t], ew_v, j, F)
                sdescs.append(pltpu.async_copy(rows[t],
                                               acc_sp.at[col_v.at[j]],
                                               ssem[t], add=True))
            for t in range(nbuf):
                sdescs[t].wait()
            return carry

        lax.fori_loop(0, nfull, quad, 0)
        if nrem:
            rdescs = []
            for t in range(nrem):
                j = nfull * nbuf + t
                rdescs.append(pltpu.async_copy(zsrc(j), rows[t], gsem[t]))
            for t in range(nrem):
                j = nfull * nbuf + t
                rdescs[t].wait()
                _scale_chunk(rows[t], ew_v, j, F)
                pltpu.sync_copy(rows[t], acc_sp.at[col_v.at[j]], add=True)
        plsc.subcore_barrier()
        for t in range(5):
            r0w = s * rpt + t * zch
            if fsplit:
                pltpu.sync_copy(acc_sp.at[pl.ds(r0w, zch)],
                                out_hbm.at[c, pl.ds(r0w, zch)])
            else:
                pltpu.sync_copy(acc_sp.at[pl.ds(r0w, zch)],
                                out_hbm.at[pl.ds(c * NHALF + r0w, zch)])

    return body


def _make_agg(F, nchunk, mode, nbuf):
    fsplit = mode == "fsplit"
    out_sh = (NC, NPAD, F) if fsplit else (NPAD, F)
    acc_rows = NPAD if fsplit else NHALF
    zch = (RPT if fsplit else NHALF // NS) // 5
    return pl.kernel(
        _make_agg_body(F, nchunk, mode, nbuf),
        out_type=jax.ShapeDtypeStruct(out_sh, jnp.float32),
        mesh=_mesh,
        compiler_params=pltpu.CompilerParams(use_tc_tiling_on_sc=False),
        scratch_types=[
            pltpu.VMEM((nchunk, K), jnp.int32),
            pltpu.VMEM((nchunk, K), jnp.int32),
            pltpu.VMEM((nchunk, K), jnp.float32),
        ] + [pltpu.VMEM((K, F), jnp.float32)] * nbuf + [
            pltpu.VMEM((zch, F), jnp.float32),
            pltpu.VMEM_SHARED((acc_rows, F), jnp.float32),
        ] + [pltpu.SemaphoreType.DMA] * (2 * nbuf),
    )


_agg128 = _make_agg(FH, NCHUNK2, "fsplit", 4)
_agg16 = _make_agg(F2, NCHUNK2, "nsplit", 8)


MBLK = 1000


def _tc_a_body(emb, dwT, db, w1aT, w1b, w1c, pn, tl, d0, d1,
               z1h_out, dis_out):
    t = jnp.tanh(jnp.dot(emb[...], dwT[...],
                         preferred_element_type=jnp.float32) + db[...])
    y = jnp.dot(t, w1aT[...], preferred_element_type=jnp.float32)
    y = y + pn[...] * w1b[...] + tl[...] * w1c[...]
    deg = d0[...] + d1[...] + 2.0
    dis = jnp.where(deg > 0, lax.rsqrt(deg), 0.0)
    z = dis * y
    z1h_out[0, :, :] = z[:, :FH]
    z1h_out[1, :, :] = z[:, FH:]
    dis_out[...] = dis


def _tc_a(emb, dwT, db, w1aT, w1b, w1c, pn, tl, d0, d1):
    return pl.pallas_call(
        _tc_a_body,
        grid=(N // MBLK,),
        in_specs=[
            pl.BlockSpec((MBLK, H), lambda i: (i, 0)),
            pl.BlockSpec((H, H), lambda i: (0, 0)),
            pl.BlockSpec((1, H), lambda i: (0, 0)),
            pl.BlockSpec((H, F1), lambda i: (0, 0)),
            pl.BlockSpec((1, F1), lambda i: (0, 0)),
            pl.BlockSpec((1, F1), lambda i: (0, 0)),
            pl.BlockSpec((MBLK, 1), lambda i: (i, 0)),
            pl.BlockSpec((MBLK, 1), lambda i: (i, 0)),
            pl.BlockSpec((MBLK, 1), lambda i: (i, 0)),
            pl.BlockSpec((MBLK, 1), lambda i: (i, 0)),
        ],
        out_specs=[
            pl.BlockSpec((NC, MBLK, FH), lambda i: (0, i, 0)),
            pl.BlockSpec((MBLK, 1), lambda i: (i, 0)),
        ],
        out_shape=[
            jax.ShapeDtypeStruct((NC, N, FH), jnp.float32),
            jax.ShapeDtypeStruct((N, 1), jnp.float32),
        ],
    )(emb, dwT, db, w1aT, w1b, w1c, pn, tl, d0, d1)


def _tc_e_body(acc, z1h, dis, b1r, w2Tp, z2_out):
    a = jnp.concatenate([acc[0], acc[1]], axis=1)
    z1 = jnp.concatenate([z1h[0], z1h[1]], axis=1)
    h = jnp.maximum(dis[...] * (a + 2.0 * z1) + b1r[...], 0.0)
    y2 = jnp.dot(h, w2Tp[...], preferred_element_type=jnp.float32)
    z2_out[...] = dis[...] * y2


def _tc_e(acc, z1h, dis, b1r, w2Tp):
    return pl.pallas_call(
        _tc_e_body,
        out_shape=jax.ShapeDtypeStruct((N, F2), jnp.float32),
    )(acc, z1h, dis, b1r, w2Tp)


def _tc_g_body(acc, z2, dis, b2p, out):
    out[...] = dis[...] * (acc[...] + 2.0 * z2[...]) + b2p[...]


def _tc_g(acc, z2, dis, b2p):
    return pl.pallas_call(
        _tc_g_body,
        out_shape=jax.ShapeDtypeStruct((N, F2), jnp.float32),
    )(acc, z2, dis, b2p)


def kernel(embedding, p_num, text_len, edge_index, edge_attr,
           dense_w, dense_b, w1, b1, w2, b2):
    row3 = edge_index[0].reshape(NT, NCHUNK, K)
    col3 = edge_index[1].reshape(NT, NCHUNK, K)
    ew3 = edge_attr.reshape(NT, NCHUNK, K)
    row2 = edge_index[0].reshape(NS, NCHUNK2, K)
    col2 = edge_index[1].reshape(NS, NCHUNK2, K)
    ew2 = edge_attr.reshape(NS, NCHUNK2, K)

    deg_parts = _deg(col3, ew3)
    d0 = deg_parts[0, :N][:, None]
    d1 = deg_parts[1, :N][:, None]

    dwT = dense_w.T
    w1aT = w1[:, :H].T
    w1b = w1[:, H][None, :]
    w1c = w1[:, H + 1][None, :]
    z1h, dis = _tc_a(embedding, dwT, dense_b[None, :], w1aT, w1b, w1c,
                     p_num, text_len, d0, d1)

    acc1 = _agg128(z1h, row2, col2, ew2)[:, :N, :]

    w2Tp = jnp.concatenate(
        [w2.T, jnp.zeros((F1, F2 - w2.shape[0]), jnp.float32)], axis=1)
    z2 = _tc_e(acc1, z1h[:, :N, :], dis, b1[None, :], w2Tp)

    acc2 = _agg16(z2, row2, col2, ew2)[:N, :]

    b2p = jnp.concatenate(
        [b2, jnp.zeros((F2 - b2.shape[0],), jnp.float32)])[None, :]
    out16 = _tc_g(acc2, z2, dis, b2p)
    return out16[:, :10]
